# Initial kernel scaffold; baseline (speedup 1.0000x reference)
import jax
import jax.numpy as jnp
from jax.experimental import pallas as pl


def kernel(x, W1, att1, b1, g_bn1, beta_bn1, W2, att2, b2, g_bn2, beta_bn2, edge_index):
    raise NotImplementedError("write your pallas kernel here")


# speedup vs baseline: 27.8635x; 27.8635x over previous
import functools

import jax
import jax.numpy as jnp
from jax import lax
from jax.experimental import pallas as pl
from jax.experimental.pallas import tpu as pltpu
from jax.experimental.pallas import tpu_sc as plsc

_N = 10000
_D = 128
_E = 320000
_H1, _O1 = 4, 128
_H2, _O2 = 8, 6
_EPS = 1e-5

_EEP = 331776
_PAD = _EEP - (_E + _N)
_ACC_ROWS = 10016
_RPT = _ACC_ROWS // 16
_FW = 48
_NP1 = 3
_ROW2 = 32
_NP2 = 2
_HPP = _H2 // _NP2
_SB = 64
_EPT1 = _EEP // 16
_EPT2 = _EEP // 32

_MESH = plsc.VectorSubcoreMesh(core_axis_name="c", subcore_axis_name="s")

_SC_PARAMS = pltpu.CompilerParams(
    needs_layout_passes=False, use_tc_tiling_on_sc=False)


def _bn1_body(x_ref, g_ref, b_ref, o_ref):
  xx = x_ref[...]
  mu = jnp.mean(xx, axis=0, keepdims=True)
  var = jnp.mean((xx - mu) * (xx - mu), axis=0, keepdims=True)
  o_ref[...] = (xx - mu) / jnp.sqrt(var + _EPS) * g_ref[...] + b_ref[...]


def _dense1a(x, g, b):
  return pl.pallas_call(
      _bn1_body,
      out_shape=jax.ShapeDtypeStruct((_N, _D), jnp.float32),
  )(x, g, b)


def _d1b_body(bn_ref, w_ref, ai_ref, aj_ref, t0_ref, t1_ref, t2_ref,
              si_ref, sj_ref):
  h = pl.program_id(0)
  bn = bn_ref[...]
  wblk = w_ref[...]
  hh = jnp.dot(bn, wblk, preferred_element_type=jnp.float32)
  t0_ref[...] = hh[:, 0:_FW][None]
  t1_ref[...] = hh[:, _FW:2 * _FW][None]
  ones = jnp.ones((_N, 1), jnp.float32)
  zeros = jnp.zeros((_N, _NP1 * _FW - _O1 - 1), jnp.float32)
  t2_ref[...] = jnp.concatenate([hh[:, 2 * _FW:_O1], ones, zeros],
                                axis=1)[None]
  ai = ai_ref[pl.ds(h, 1), :]
  aj = aj_ref[pl.ds(h, 1), :]
  dims = (((1,), (1,)), ((), ()))
  si_ref[...] = lax.dot_general(ai, hh, dims,
                                preferred_element_type=jnp.float32)[None]
  sj_ref[...] = lax.dot_general(aj, hh, dims,
                                preferred_element_type=jnp.float32)[None]


def _dense1b(bn, w1, atti, attj):
  tspec = pl.BlockSpec((1, _N, _FW), lambda h: (h, 0, 0))
  tshape = jax.ShapeDtypeStruct((_H1, _N, _FW), jnp.float32)
  return pl.pallas_call(
      _d1b_body,
      grid=(_H1,),
      in_specs=[
          pl.BlockSpec((_N, _D), lambda h: (0, 0)),
          pl.BlockSpec((_D, _O1), lambda h: (0, h)),
          pl.BlockSpec((_H1, _O1), lambda h: (0, 0)),
          pl.BlockSpec((_H1, _O1), lambda h: (0, 0)),
      ],
      out_specs=[
          tspec, tspec, tspec,
          pl.BlockSpec((1, 1, _N), lambda h: (h, 0, 0)),
          pl.BlockSpec((1, 1, _N), lambda h: (h, 0, 0)),
      ],
      out_shape=[
          tshape, tshape, tshape,
          jax.ShapeDtypeStruct((_H1, 1, _N), jnp.float32),
          jax.ShapeDtypeStruct((_H1, 1, _N), jnp.float32),
      ],
  )(bn, w1, atti, attj)


def _d2a_body(acc_ref, b1_ref, o_ref, mu_ref, msq_ref):
  tail = _O1 - 2 * _FW
  num = jnp.concatenate(
      [acc_ref[0, 0:_N, :], acc_ref[1, 0:_N, :], acc_ref[2, 0:_N, 0:tail]],
      axis=1)
  den = acc_ref[2, 0:_N, tail:tail + 1]
  o = num / (den + 1e-16) + b1_ref[...]
  o_ref[...] = o
  mu_ref[...] = jnp.mean(o, axis=0, keepdims=True)[None]
  msq_ref[...] = jnp.mean(o * o, axis=0, keepdims=True)[None]


def _dense2a(acc1, b1):
  return pl.pallas_call(
      _d2a_body,
      grid=(_H1,),
      in_specs=[
          pl.BlockSpec((_NP1, _ACC_ROWS, _FW), lambda h: (h, 0, 0)),
          pl.BlockSpec((1, _O1), lambda h: (0, h)),
      ],
      out_specs=[
          pl.BlockSpec((_N, _O1), lambda h: (0, h)),
          pl.BlockSpec((1, 1, _O1), lambda h: (h, 0, 0)),
          pl.BlockSpec((1, 1, _O1), lambda h: (h, 0, 0)),
      ],
      out_shape=[
          jax.ShapeDtypeStruct((_N, _H1 * _O1), jnp.float32),
          jax.ShapeDtypeStruct((_H1, 1, _O1), jnp.float32),
          jax.ShapeDtypeStruct((_H1, 1, _O1), jnp.float32),
      ],
  )(acc1, b1)


def _d2b1_body(o1_ref, mu_ref, msq_ref, g_ref, be_ref, w2_ref, h2_ref):
  mu = mu_ref[...]
  var = msq_ref[...] - mu * mu
  s = g_ref[...] / jnp.sqrt(var + _EPS)
  off = be_ref[...] - mu * s
  bn2 = o1_ref[...] * s + off
  part = jnp.dot(bn2, w2_ref[...], preferred_element_type=jnp.float32)

  @pl.when(pl.program_id(0) == 0)
  def _():
    h2_ref[...] = jnp.zeros_like(h2_ref)

  h2_ref[...] += part


def _dense2b1(out1, mu, msq, g2, be2, w2):
  return pl.pallas_call(
      _d2b1_body,
      grid=(_H1,),
      in_specs=[
          pl.BlockSpec((_N, _O1), lambda k: (0, k)),
          pl.BlockSpec((1, _O1), lambda k: (0, k)),
          pl.BlockSpec((1, _O1), lambda k: (0, k)),
          pl.BlockSpec((1, _O1), lambda k: (0, k)),
          pl.BlockSpec((1, _O1), lambda k: (0, k)),
          pl.BlockSpec((_O1, _H2 * _O2), lambda k: (k, 0)),
      ],
      out_specs=pl.BlockSpec((_N, _H2 * _O2), lambda k: (0, 0)),
      out_shape=jax.ShapeDtypeStruct((_N, _H2 * _O2), jnp.float32),
  )(out1, mu, msq, g2, be2, w2)


def _d2b2_body(h2_ref, att_ref, ha_ref, hb_ref, t_ref):
  h2 = h2_ref[...]
  ones = jnp.ones((_N, 1), jnp.float32)
  zeros = jnp.zeros((_N, 1), jnp.float32)
  for p, ref in ((0, ha_ref), (1, hb_ref)):
    parts = []
    for hh in range(_HPP):
      h = p * _HPP + hh
      parts.extend([h2[:, 6 * h:6 * h + 6], ones, zeros])
    ref[...] = jnp.concatenate(parts, axis=1)
  att = att_ref[0]
  wlr = att[:, 0:_O2] + att[:, _O2:2 * _O2]
  dims = (((1,), (1,)), ((), ()))
  rows = []
  for h in range(_H2):
    rows.append(lax.dot_general(wlr[h:h + 1, :], h2[:, 6 * h:6 * h + 6], dims,
                                preferred_element_type=jnp.float32))
  t_ref[...] = jnp.tanh(jnp.concatenate(rows, axis=0))


def _dense2b(out1, mu, msq, g2, be2, w2, att2):
  h2 = _dense2b1(out1, mu, msq, g2, be2, w2)
  return pl.pallas_call(
      _d2b2_body,
      out_shape=[
          jax.ShapeDtypeStruct((_N, _ROW2), jnp.float32),
          jax.ShapeDtypeStruct((_N, _ROW2), jnp.float32),
          jax.ShapeDtypeStruct((_H2, _N), jnp.float32),
      ],
  )(h2, att2)


def _final_body(a_ref, b2_ref, o_ref):
  p = pl.program_id(0)
  a = a_ref[0, 0, 0:_N, :] + a_ref[1, 0, 0:_N, :]
  col = lax.broadcasted_iota(jnp.int32, (_ROW2, _ROW2), 0)
  dst = lax.broadcasted_iota(jnp.int32, (_ROW2, _ROW2), 1)
  sel = jnp.where((col // 8 == dst // 8) & (col % 8 == 6), 1.0, 0.0)
  den = jnp.dot(a, sel, preferred_element_type=jnp.float32) + 1e-16
  r = a / den
  colp = lax.broadcasted_iota(jnp.int32, (_ROW2, _O2), 0)
  dstp = lax.broadcasted_iota(jnp.int32, (_ROW2, _O2), 1)
  proj = jnp.where(colp % 8 == dstp, 1.0, 0.0)
  m_p = jnp.dot(r, proj, preferred_element_type=jnp.float32)

  @pl.when(p == 0)
  def _():
    o_ref[...] = jnp.zeros_like(o_ref)

  o_ref[...] += m_p

  @pl.when(p == _NP2 - 1)
  def _():
    m = o_ref[...] * (1.0 / _H2) + b2_ref[...]
    m = jnp.where(m > 0, m, jnp.exp(m) - 1.0)
    z = m - jnp.max(m, axis=1, keepdims=True)
    o_ref[...] = z - jnp.log(jnp.sum(jnp.exp(z), axis=1, keepdims=True))


def _final(acc2, b2):
  return pl.pallas_call(
      _final_body,
      grid=(_NP2,),
      in_specs=[
          pl.BlockSpec((2, 1, _ACC_ROWS, _ROW2), lambda p: (0, p, 0, 0)),
          pl.BlockSpec((1, _O2), lambda p: (0, 0)),
      ],
      out_specs=pl.BlockSpec((_N, _O2), lambda p: (0, 0)),
      out_shape=jax.ShapeDtypeStruct((_N, _O2), jnp.float32),
  )(acc2, b2)


def _edge_mask(e_base, off, src16, dst16):
  ge = e_base + off + lax.iota(jnp.int32, 16)
  to_dump = jnp.logical_and(ge < _E, src16 == dst16)
  return jnp.where(to_dump, jnp.int32(_N), dst16)


def _zero_acc(zrow_hbm, rows_v, acc, s):
  pltpu.sync_copy(zrow_hbm, rows_v.at[0])
  for k in range(_RPT // _SB):
    pltpu.sync_copy(rows_v.at[0], acc.at[pl.ds(s * _RPT + k * _SB, _SB)])
  rem = _RPT % _SB
  if rem:
    pltpu.sync_copy(rows_v.at[0, pl.ds(0, rem)],
                    acc.at[pl.ds(s * _RPT + (_RPT // _SB) * _SB, rem)])


def _read_acc(acc, out_slice, rows_v, s):
  nfull = _RPT // _SB
  for k in range(nfull):
    pltpu.sync_copy(acc.at[pl.ds(s * _RPT + k * _SB, _SB)], rows_v.at[0])
    pltpu.sync_copy(rows_v.at[0], out_slice.at[pl.ds(s * _RPT + k * _SB, _SB)])
  rem = _RPT % _SB
  if rem:
    base = s * _RPT + nfull * _SB
    pltpu.sync_copy(acc.at[pl.ds(base, rem)], rows_v.at[0, pl.ds(0, rem)])
    pltpu.sync_copy(rows_v.at[0, pl.ds(0, rem)], out_slice.at[pl.ds(base, rem)])


def _edge1_body(si_hbm, sj_hbm, src_hbm, dst_hbm, t0_hbm, t1_hbm, t2_hbm,
                zrow_hbm, out_hbm, src_v, dst_v, si_v, sj_v, wc_v, ri_v,
                db_v, rows_v, acc, gsem0, gsem1):
  c = lax.axis_index("c")
  s = lax.axis_index("s")
  e_base = s * _EPT1
  pltpu.sync_copy(src_hbm.at[pl.ds(e_base, _EPT1)], src_v)
  pltpu.sync_copy(dst_hbm.at[pl.ds(e_base, _EPT1)], dst_v)
  tables = (t0_hbm, t1_hbm, t2_hbm)
  gsems = (gsem0, gsem1)

  @pl.loop(0, 2)
  def _heads(hp):
    h = c * 2 + hp
    pltpu.sync_copy(si_hbm.at[h], si_v)
    pltpu.sync_copy(sj_hbm.at[h], sj_v)
    row_base = h * _N
    for p in range(_NP1):
      _zero_acc(zrow_hbm, rows_v, acc, s)
      plsc.subcore_barrier()
      table = tables[p]

      @pl.loop(0, _EPT1, step=2 * _SB)
      def _pair(e0):
        copies = []
        for b in range(2):
          off0 = e0 + b * _SB
          for g in range(_SB // 16):
            off = off0 + g * 16
            src16 = src_v[pl.ds(off, 16)]
            dst16 = _edge_mask(e_base, off, src16, dst_v[pl.ds(off, 16)])
            if p == 0:
              dse16 = jnp.where(dst16 == _N, src16, dst16)
              z = (plsc.load_gather(si_v, [dse16]) +
                   plsc.load_gather(sj_v, [src16]))
              z = jnp.where(z > 0, z, 0.2 * z)
              wc_v[pl.ds(off, 16)] = jnp.exp(z)
            ri_v[b, pl.ds(g * 16, 16)] = src16 + row_base
            db_v[b, pl.ds(g * 16, 16)] = dst16
          copies.append(
              pltpu.async_copy(table.at[ri_v.at[b]], rows_v.at[b],
                               gsems[b]))
        for b in range(2):
          copies[b].wait()

          @pl.loop(0, _SB, step=8)
          def _scale(es):
            for ee in range(8):
              e = es + ee
              ws = plsc.load_gather(
                  wc_v, [jnp.full((16,), e0 + b * _SB + e, jnp.int32)])
              for cc in range(_FW // 16):
                rows_v[b, e, pl.ds(cc * 16, 16)] = (
                    rows_v[b, e, pl.ds(cc * 16, 16)] * ws)

          pltpu.sync_copy(rows_v.at[b], acc.at[db_v.at[b]], add=True)

      plsc.subcore_barrier()
      _read_acc(acc, out_hbm.at[h * _NP1 + p], rows_v, s)
      plsc.subcore_barrier()


@functools.partial(
    pl.kernel,
    out_type=jax.ShapeDtypeStruct((_H1 * _NP1, _ACC_ROWS, _FW), jnp.float32),
    mesh=_MESH,
    compiler_params=_SC_PARAMS,
    scratch_types=[
        pltpu.VMEM((_EPT1,), jnp.int32),
        pltpu.VMEM((_EPT1,), jnp.int32),
        pltpu.VMEM((_N,), jnp.float32),
        pltpu.VMEM((_N,), jnp.float32),
        pltpu.VMEM((_EPT1,), jnp.float32),
        pltpu.VMEM((2, _SB), jnp.int32),
        pltpu.VMEM((2, _SB), jnp.int32),
        pltpu.VMEM((2, _SB, _FW), jnp.float32),
        pltpu.VMEM_SHARED((_ACC_ROWS, _FW), jnp.float32),
        pltpu.SemaphoreType.DMA,
        pltpu.SemaphoreType.DMA,
    ],
)
def _edge1(*refs):
  _edge1_body(*refs)


def _edge2_body(t_hbm, src_hbm, dst_hbm, ha_hbm, hb_hbm, zrow_hbm, out_hbm,
                src_v, dst_v, t_v, w_v, ri_v, db_v, rows_v, acc, gsem0,
                gsem1):
  c = lax.axis_index("c")
  s = lax.axis_index("s")
  e_base = (c * 16 + s) * _EPT2
  pltpu.sync_copy(src_hbm.at[pl.ds(e_base, _EPT2)], src_v)
  pltpu.sync_copy(dst_hbm.at[pl.ds(e_base, _EPT2)], dst_v)
  pltpu.sync_copy(t_hbm, t_v)
  tables = (ha_hbm, hb_hbm)
  gsems = (gsem0, gsem1)
  for p in range(_NP2):
    _zero_acc(zrow_hbm, rows_v, acc, s)
    plsc.subcore_barrier()
    table = tables[p]

    @pl.loop(0, _EPT2, step=2 * _SB)
    def _pair(e0):
      copies = []
      for b in range(2):
        off0 = e0 + b * _SB
        for g in range(_SB // 16):
          off = off0 + g * 16
          src16 = src_v[pl.ds(off, 16)]
          dst16 = _edge_mask(e_base, off, src16, dst_v[pl.ds(off, 16)])
          for hh in range(_HPP):
            th = plsc.load_gather(
                t_v, [jnp.full((16,), p * _HPP + hh, jnp.int32), src16])
            w_v[b, hh, pl.ds(g * 16, 16)] = jnp.exp(th)
          ri_v[b, pl.ds(g * 16, 16)] = src16
          db_v[b, pl.ds(g * 16, 16)] = dst16
        copies.append(
            pltpu.async_copy(table.at[ri_v.at[b]], rows_v.at[b], gsems[b]))
      for b in range(2):
        copies[b].wait()

        @pl.loop(0, _SB, step=8)
        def _scale(es):
          for ee in range(8):
            e = es + ee
            for k in range(_ROW2 // 16):
              idx_h = 2 * k + lax.shift_right_logical(
                  lax.iota(jnp.int32, 16), 3)
              wrep = plsc.load_gather(
                  w_v.at[b], [idx_h, jnp.full((16,), e, jnp.int32)])
              rows_v[b, e, pl.ds(k * 16, 16)] = (
                  rows_v[b, e, pl.ds(k * 16, 16)] * wrep)

        pltpu.sync_copy(rows_v.at[b], acc.at[db_v.at[b]], add=True)

    plsc.subcore_barrier()
    _read_acc(acc, out_hbm.at[c, p], rows_v, s)
    plsc.subcore_barrier()


@functools.partial(
    pl.kernel,
    out_type=jax.ShapeDtypeStruct((2, _NP2, _ACC_ROWS, _ROW2), jnp.float32),
    mesh=_MESH,
    compiler_params=_SC_PARAMS,
    scratch_types=[
        pltpu.VMEM((_EPT2,), jnp.int32),
        pltpu.VMEM((_EPT2,), jnp.int32),
        pltpu.VMEM((_H2, _N), jnp.float32),
        pltpu.VMEM((2, _HPP, _SB), jnp.float32),
        pltpu.VMEM((2, _SB), jnp.int32),
        pltpu.VMEM((2, _SB), jnp.int32),
        pltpu.VMEM((2, _SB, _ROW2), jnp.float32),
        pltpu.VMEM_SHARED((_ACC_ROWS, _ROW2), jnp.float32),
        pltpu.SemaphoreType.DMA,
        pltpu.SemaphoreType.DMA,
    ],
)
def _edge2(*refs):
  _edge2_body(*refs)


def kernel(x, W1, att1, b1, g_bn1, beta_bn1, W2, att2, b2, g_bn2, beta_bn2,
           edge_index):
  loops = jnp.arange(_N, dtype=jnp.int32)
  src = jnp.concatenate(
      [edge_index[0], loops, jnp.zeros((_PAD,), jnp.int32)])
  dst = jnp.concatenate(
      [edge_index[1], loops, jnp.full((_PAD,), _N, jnp.int32)])
  zrow1 = jnp.zeros((_SB, _FW), jnp.float32)
  zrow2 = jnp.zeros((_SB, _ROW2), jnp.float32)

  bn1 = _dense1a(x, g_bn1.reshape(1, _D), beta_bn1.reshape(1, _D))
  atti = att1[0, :, :_O1]
  attj = att1[0, :, _O1:]
  t0, t1, t2, si3, sj3 = _dense1b(bn1, W1, atti, attj)
  acc1 = _edge1(si3.reshape(_H1, _N), sj3.reshape(_H1, _N), src, dst,
                t0.reshape(_H1 * _N, _FW), t1.reshape(_H1 * _N, _FW),
                t2.reshape(_H1 * _N, _FW), zrow1)
  out1, mu3, msq3 = _dense2a(acc1, b1.reshape(1, _H1 * _O1))
  ha, hb, t_hm = _dense2b(out1, mu3.reshape(1, _H1 * _O1),
                          msq3.reshape(1, _H1 * _O1),
                          g_bn2.reshape(1, _H1 * _O1),
                          beta_bn2.reshape(1, _H1 * _O1), W2, att2)
  acc2 = _edge2(t_hm, src, dst, ha, hb, zrow2)
  return _final(acc2, b2.reshape(1, _O2))


# speedup vs baseline: 27.9069x; 1.0016x over previous
import functools

import jax
import jax.numpy as jnp
from jax import lax
from jax.experimental import pallas as pl
from jax.experimental.pallas import tpu as pltpu
from jax.experimental.pallas import tpu_sc as plsc

_N = 10000
_D = 128
_E = 320000
_H1, _O1 = 4, 128
_H2, _O2 = 8, 6
_EPS = 1e-5

_EEP = 335872
_PAD = _EEP - (_E + _N)
_ACC_ROWS = 10016
_RPT = _ACC_ROWS // 16
_FW = 48
_NP1 = 3
_ROW2 = 32
_NP2 = 2
_HPP = _H2 // _NP2
_SB = 128
_EPT1 = _EEP // 16
_EPT2 = _EEP // 32

_MESH = plsc.VectorSubcoreMesh(core_axis_name="c", subcore_axis_name="s")

_SC_PARAMS = pltpu.CompilerParams(
    needs_layout_passes=False, use_tc_tiling_on_sc=False)


def _bn1_body(x_ref, g_ref, b_ref, o_ref):
  xx = x_ref[...]
  mu = jnp.mean(xx, axis=0, keepdims=True)
  var = jnp.mean((xx - mu) * (xx - mu), axis=0, keepdims=True)
  o_ref[...] = (xx - mu) / jnp.sqrt(var + _EPS) * g_ref[...] + b_ref[...]


def _dense1a(x, g, b):
  return pl.pallas_call(
      _bn1_body,
      out_shape=jax.ShapeDtypeStruct((_N, _D), jnp.float32),
  )(x, g, b)


def _d1b_body(bn_ref, w_ref, ai_ref, aj_ref, t0_ref, t1_ref, t2_ref,
              si_ref, sj_ref):
  h = pl.program_id(0)
  bn = bn_ref[...]
  wblk = w_ref[...]
  hh = jnp.dot(bn, wblk, preferred_element_type=jnp.float32)
  t0_ref[...] = hh[:, 0:_FW][None]
  t1_ref[...] = hh[:, _FW:2 * _FW][None]
  ones = jnp.ones((_N, 1), jnp.float32)
  zeros = jnp.zeros((_N, _NP1 * _FW - _O1 - 1), jnp.float32)
  t2_ref[...] = jnp.concatenate([hh[:, 2 * _FW:_O1], ones, zeros],
                                axis=1)[None]
  ai = ai_ref[pl.ds(h, 1), :]
  aj = aj_ref[pl.ds(h, 1), :]
  dims = (((1,), (1,)), ((), ()))
  si_ref[...] = lax.dot_general(ai, hh, dims,
                                preferred_element_type=jnp.float32)[None]
  sj_ref[...] = lax.dot_general(aj, hh, dims,
                                preferred_element_type=jnp.float32)[None]


def _dense1b(bn, w1, atti, attj):
  tspec = pl.BlockSpec((1, _N, _FW), lambda h: (h, 0, 0))
  tshape = jax.ShapeDtypeStruct((_H1, _N, _FW), jnp.float32)
  return pl.pallas_call(
      _d1b_body,
      grid=(_H1,),
      in_specs=[
          pl.BlockSpec((_N, _D), lambda h: (0, 0)),
          pl.BlockSpec((_D, _O1), lambda h: (0, h)),
          pl.BlockSpec((_H1, _O1), lambda h: (0, 0)),
          pl.BlockSpec((_H1, _O1), lambda h: (0, 0)),
      ],
      out_specs=[
          tspec, tspec, tspec,
          pl.BlockSpec((1, 1, _N), lambda h: (h, 0, 0)),
          pl.BlockSpec((1, 1, _N), lambda h: (h, 0, 0)),
      ],
      out_shape=[
          tshape, tshape, tshape,
          jax.ShapeDtypeStruct((_H1, 1, _N), jnp.float32),
          jax.ShapeDtypeStruct((_H1, 1, _N), jnp.float32),
      ],
  )(bn, w1, atti, attj)


def _d2a_body(acc_ref, b1_ref, o_ref, mu_ref, msq_ref):
  tail = _O1 - 2 * _FW
  num = jnp.concatenate(
      [acc_ref[0, 0:_N, :], acc_ref[1, 0:_N, :], acc_ref[2, 0:_N, 0:tail]],
      axis=1)
  den = acc_ref[2, 0:_N, tail:tail + 1]
  o = num / (den + 1e-16) + b1_ref[...]
  o_ref[...] = o
  mu_ref[...] = jnp.mean(o, axis=0, keepdims=True)[None]
  msq_ref[...] = jnp.mean(o * o, axis=0, keepdims=True)[None]


def _dense2a(acc1, b1):
  return pl.pallas_call(
      _d2a_body,
      grid=(_H1,),
      in_specs=[
          pl.BlockSpec((_NP1, _ACC_ROWS, _FW), lambda h: (h, 0, 0)),
          pl.BlockSpec((1, _O1), lambda h: (0, h)),
      ],
      out_specs=[
          pl.BlockSpec((_N, _O1), lambda h: (0, h)),
          pl.BlockSpec((1, 1, _O1), lambda h: (h, 0, 0)),
          pl.BlockSpec((1, 1, _O1), lambda h: (h, 0, 0)),
      ],
      out_shape=[
          jax.ShapeDtypeStruct((_N, _H1 * _O1), jnp.float32),
          jax.ShapeDtypeStruct((_H1, 1, _O1), jnp.float32),
          jax.ShapeDtypeStruct((_H1, 1, _O1), jnp.float32),
      ],
  )(acc1, b1)


def _d2b1_body(o1_ref, mu_ref, msq_ref, g_ref, be_ref, w2_ref, h2_ref):
  mu = mu_ref[...]
  var = msq_ref[...] - mu * mu
  s = g_ref[...] / jnp.sqrt(var + _EPS)
  off = be_ref[...] - mu * s
  bn2 = o1_ref[...] * s + off
  part = jnp.dot(bn2, w2_ref[...], preferred_element_type=jnp.float32)

  @pl.when(pl.program_id(0) == 0)
  def _():
    h2_ref[...] = jnp.zeros_like(h2_ref)

  h2_ref[...] += part


def _dense2b1(out1, mu, msq, g2, be2, w2):
  return pl.pallas_call(
      _d2b1_body,
      grid=(_H1,),
      in_specs=[
          pl.BlockSpec((_N, _O1), lambda k: (0, k)),
          pl.BlockSpec((1, _O1), lambda k: (0, k)),
          pl.BlockSpec((1, _O1), lambda k: (0, k)),
          pl.BlockSpec((1, _O1), lambda k: (0, k)),
          pl.BlockSpec((1, _O1), lambda k: (0, k)),
          pl.BlockSpec((_O1, _H2 * _O2), lambda k: (k, 0)),
      ],
      out_specs=pl.BlockSpec((_N, _H2 * _O2), lambda k: (0, 0)),
      out_shape=jax.ShapeDtypeStruct((_N, _H2 * _O2), jnp.float32),
  )(out1, mu, msq, g2, be2, w2)


def _d2b2_body(h2_ref, att_ref, ha_ref, hb_ref, t_ref):
  h2 = h2_ref[...]
  ones = jnp.ones((_N, 1), jnp.float32)
  zeros = jnp.zeros((_N, 1), jnp.float32)
  for p, ref in ((0, ha_ref), (1, hb_ref)):
    parts = []
    for hh in range(_HPP):
      h = p * _HPP + hh
      parts.extend([h2[:, 6 * h:6 * h + 6], ones, zeros])
    ref[...] = jnp.concatenate(parts, axis=1)
  att = att_ref[0]
  wlr = att[:, 0:_O2] + att[:, _O2:2 * _O2]
  dims = (((1,), (1,)), ((), ()))
  rows = []
  for h in range(_H2):
    rows.append(lax.dot_general(wlr[h:h + 1, :], h2[:, 6 * h:6 * h + 6], dims,
                                preferred_element_type=jnp.float32))
  t_ref[...] = jnp.tanh(jnp.concatenate(rows, axis=0))


def _dense2b(out1, mu, msq, g2, be2, w2, att2):
  h2 = _dense2b1(out1, mu, msq, g2, be2, w2)
  return pl.pallas_call(
      _d2b2_body,
      out_shape=[
          jax.ShapeDtypeStruct((_N, _ROW2), jnp.float32),
          jax.ShapeDtypeStruct((_N, _ROW2), jnp.float32),
          jax.ShapeDtypeStruct((_H2, _N), jnp.float32),
      ],
  )(h2, att2)


def _final_body(a_ref, b2_ref, o_ref):
  p = pl.program_id(0)
  a = a_ref[0, 0, 0:_N, :] + a_ref[1, 0, 0:_N, :]
  col = lax.broadcasted_iota(jnp.int32, (_ROW2, _ROW2), 0)
  dst = lax.broadcasted_iota(jnp.int32, (_ROW2, _ROW2), 1)
  sel = jnp.where((col // 8 == dst // 8) & (col % 8 == 6), 1.0, 0.0)
  den = jnp.dot(a, sel, preferred_element_type=jnp.float32) + 1e-16
  r = a / den
  colp = lax.broadcasted_iota(jnp.int32, (_ROW2, _O2), 0)
  dstp = lax.broadcasted_iota(jnp.int32, (_ROW2, _O2), 1)
  proj = jnp.where(colp % 8 == dstp, 1.0, 0.0)
  m_p = jnp.dot(r, proj, preferred_element_type=jnp.float32)

  @pl.when(p == 0)
  def _():
    o_ref[...] = jnp.zeros_like(o_ref)

  o_ref[...] += m_p

  @pl.when(p == _NP2 - 1)
  def _():
    m = o_ref[...] * (1.0 / _H2) + b2_ref[...]
    m = jnp.where(m > 0, m, jnp.exp(m) - 1.0)
    z = m - jnp.max(m, axis=1, keepdims=True)
    o_ref[...] = z - jnp.log(jnp.sum(jnp.exp(z), axis=1, keepdims=True))


def _final(acc2, b2):
  return pl.pallas_call(
      _final_body,
      grid=(_NP2,),
      in_specs=[
          pl.BlockSpec((2, 1, _ACC_ROWS, _ROW2), lambda p: (0, p, 0, 0)),
          pl.BlockSpec((1, _O2), lambda p: (0, 0)),
      ],
      out_specs=pl.BlockSpec((_N, _O2), lambda p: (0, 0)),
      out_shape=jax.ShapeDtypeStruct((_N, _O2), jnp.float32),
  )(acc2, b2)


def _edge_mask(e_base, off, src16, dst16):
  ge = e_base + off + lax.iota(jnp.int32, 16)
  to_dump = jnp.logical_and(ge < _E, src16 == dst16)
  return jnp.where(to_dump, jnp.int32(_N), dst16)


def _zero_acc(zrow_hbm, rows_v, acc, s):
  pltpu.sync_copy(zrow_hbm, rows_v.at[0])
  for k in range(_RPT // _SB):
    pltpu.sync_copy(rows_v.at[0], acc.at[pl.ds(s * _RPT + k * _SB, _SB)])
  rem = _RPT % _SB
  if rem:
    pltpu.sync_copy(rows_v.at[0, pl.ds(0, rem)],
                    acc.at[pl.ds(s * _RPT + (_RPT // _SB) * _SB, rem)])


def _read_acc(acc, out_slice, rows_v, s):
  nfull = _RPT // _SB
  for k in range(nfull):
    pltpu.sync_copy(acc.at[pl.ds(s * _RPT + k * _SB, _SB)], rows_v.at[0])
    pltpu.sync_copy(rows_v.at[0], out_slice.at[pl.ds(s * _RPT + k * _SB, _SB)])
  rem = _RPT % _SB
  if rem:
    base = s * _RPT + nfull * _SB
    pltpu.sync_copy(acc.at[pl.ds(base, rem)], rows_v.at[0, pl.ds(0, rem)])
    pltpu.sync_copy(rows_v.at[0, pl.ds(0, rem)], out_slice.at[pl.ds(base, rem)])


def _edge1_body(si_hbm, sj_hbm, src_hbm, dst_hbm, t0_hbm, t1_hbm, t2_hbm,
                zrow_hbm, out_hbm, src_v, dst_v, si_v, sj_v, wc_v, ri_v,
                db_v, rows_v, acc, gsem0, gsem1, ssem0, ssem1):
  c = lax.axis_index("c")
  s = lax.axis_index("s")
  e_base = s * _EPT1
  pltpu.sync_copy(src_hbm.at[pl.ds(e_base, _EPT1)], src_v)
  pltpu.sync_copy(dst_hbm.at[pl.ds(e_base, _EPT1)], dst_v)
  tables = (t0_hbm, t1_hbm, t2_hbm)
  gsems = (gsem0, gsem1)
  ssems = (ssem0, ssem1)

  @pl.loop(0, 2)
  def _heads(hp):
    h = c * 2 + hp
    pltpu.sync_copy(si_hbm.at[h], si_v)
    pltpu.sync_copy(sj_hbm.at[h], sj_v)
    row_base = h * _N
    for p in range(_NP1):
      _zero_acc(zrow_hbm, rows_v, acc, s)
      plsc.subcore_barrier()
      table = tables[p]

      @pl.loop(0, _EPT1, step=2 * _SB)
      def _pair(e0):
        copies = []
        for b in range(2):
          off0 = e0 + b * _SB
          for g in range(_SB // 16):
            off = off0 + g * 16
            src16 = src_v[pl.ds(off, 16)]
            dst16 = _edge_mask(e_base, off, src16, dst_v[pl.ds(off, 16)])
            if p == 0:
              dse16 = jnp.where(dst16 == _N, src16, dst16)
              z = (plsc.load_gather(si_v, [dse16]) +
                   plsc.load_gather(sj_v, [src16]))
              z = jnp.where(z > 0, z, 0.2 * z)
              wc_v[pl.ds(off, 16)] = jnp.exp(z)
            ri_v[b, pl.ds(g * 16, 16)] = src16 + row_base
            db_v[b, pl.ds(g * 16, 16)] = dst16
          copies.append(
              pltpu.async_copy(table.at[ri_v.at[b]], rows_v.at[b],
                               gsems[b]))
        scats = []
        for b in range(2):
          copies[b].wait()

          @pl.loop(0, _SB, step=8)
          def _scale(es):
            for ee in range(8):
              e = es + ee
              ws = plsc.load_gather(
                  wc_v, [jnp.full((16,), e0 + b * _SB + e, jnp.int32)])
              for cc in range(_FW // 16):
                rows_v[b, e, pl.ds(cc * 16, 16)] = (
                    rows_v[b, e, pl.ds(cc * 16, 16)] * ws)

          scats.append(
              pltpu.async_copy(rows_v.at[b], acc.at[db_v.at[b]], ssems[b],
                               add=True))
        for b in range(2):
          scats[b].wait()

      plsc.subcore_barrier()
      _read_acc(acc, out_hbm.at[h * _NP1 + p], rows_v, s)
      plsc.subcore_barrier()


@functools.partial(
    pl.kernel,
    out_type=jax.ShapeDtypeStruct((_H1 * _NP1, _ACC_ROWS, _FW), jnp.float32),
    mesh=_MESH,
    compiler_params=_SC_PARAMS,
    scratch_types=[
        pltpu.VMEM((_EPT1,), jnp.int32),
        pltpu.VMEM((_EPT1,), jnp.int32),
        pltpu.VMEM((_N,), jnp.float32),
        pltpu.VMEM((_N,), jnp.float32),
        pltpu.VMEM((_EPT1,), jnp.float32),
        pltpu.VMEM((2, _SB), jnp.int32),
        pltpu.VMEM((2, _SB), jnp.int32),
        pltpu.VMEM((2, _SB, _FW), jnp.float32),
        pltpu.VMEM_SHARED((_ACC_ROWS, _FW), jnp.float32),
        pltpu.SemaphoreType.DMA,
        pltpu.SemaphoreType.DMA,
        pltpu.SemaphoreType.DMA,
        pltpu.SemaphoreType.DMA,
    ],
)
def _edge1(*refs):
  _edge1_body(*refs)


def _edge2_body(t_hbm, src_hbm, dst_hbm, ha_hbm, hb_hbm, zrow_hbm, out_hbm,
                src_v, dst_v, t_v, w_v, ri_v, db_v, rows_v, acc, gsem0,
                gsem1, ssem0, ssem1):
  c = lax.axis_index("c")
  s = lax.axis_index("s")
  e_base = (c * 16 + s) * _EPT2
  pltpu.sync_copy(src_hbm.at[pl.ds(e_base, _EPT2)], src_v)
  pltpu.sync_copy(dst_hbm.at[pl.ds(e_base, _EPT2)], dst_v)
  pltpu.sync_copy(t_hbm, t_v)
  tables = (ha_hbm, hb_hbm)
  gsems = (gsem0, gsem1)
  ssems = (ssem0, ssem1)
  for p in range(_NP2):
    _zero_acc(zrow_hbm, rows_v, acc, s)
    plsc.subcore_barrier()
    table = tables[p]

    @pl.loop(0, _EPT2, step=2 * _SB)
    def _pair(e0):
      copies = []
      for b in range(2):
        off0 = e0 + b * _SB
        for g in range(_SB // 16):
          off = off0 + g * 16
          src16 = src_v[pl.ds(off, 16)]
          dst16 = _edge_mask(e_base, off, src16, dst_v[pl.ds(off, 16)])
          for hh in range(_HPP):
            th = plsc.load_gather(
                t_v, [jnp.full((16,), p * _HPP + hh, jnp.int32), src16])
            w_v[b, hh, pl.ds(g * 16, 16)] = jnp.exp(th)
          ri_v[b, pl.ds(g * 16, 16)] = src16
          db_v[b, pl.ds(g * 16, 16)] = dst16
        copies.append(
            pltpu.async_copy(table.at[ri_v.at[b]], rows_v.at[b], gsems[b]))
      scats = []
      for b in range(2):
        copies[b].wait()

        @pl.loop(0, _SB, step=8)
        def _scale(es):
          for ee in range(8):
            e = es + ee
            for k in range(_ROW2 // 16):
              idx_h = 2 * k + lax.shift_right_logical(
                  lax.iota(jnp.int32, 16), 3)
              wrep = plsc.load_gather(
                  w_v.at[b], [idx_h, jnp.full((16,), e, jnp.int32)])
              rows_v[b, e, pl.ds(k * 16, 16)] = (
                  rows_v[b, e, pl.ds(k * 16, 16)] * wrep)

        scats.append(
            pltpu.async_copy(rows_v.at[b], acc.at[db_v.at[b]], ssems[b],
                             add=True))
      for b in range(2):
        scats[b].wait()

    plsc.subcore_barrier()
    _read_acc(acc, out_hbm.at[c, p], rows_v, s)
    plsc.subcore_barrier()


@functools.partial(
    pl.kernel,
    out_type=jax.ShapeDtypeStruct((2, _NP2, _ACC_ROWS, _ROW2), jnp.float32),
    mesh=_MESH,
    compiler_params=_SC_PARAMS,
    scratch_types=[
        pltpu.VMEM((_EPT2,), jnp.int32),
        pltpu.VMEM((_EPT2,), jnp.int32),
        pltpu.VMEM((_H2, _N), jnp.float32),
        pltpu.VMEM((2, _HPP, _SB), jnp.float32),
        pltpu.VMEM((2, _SB), jnp.int32),
        pltpu.VMEM((2, _SB), jnp.int32),
        pltpu.VMEM((2, _SB, _ROW2), jnp.float32),
        pltpu.VMEM_SHARED((_ACC_ROWS, _ROW2), jnp.float32),
        pltpu.SemaphoreType.DMA,
        pltpu.SemaphoreType.DMA,
        pltpu.SemaphoreType.DMA,
        pltpu.SemaphoreType.DMA,
    ],
)
def _edge2(*refs):
  _edge2_body(*refs)


def kernel(x, W1, att1, b1, g_bn1, beta_bn1, W2, att2, b2, g_bn2, beta_bn2,
           edge_index):
  loops = jnp.arange(_N, dtype=jnp.int32)
  src = jnp.concatenate(
      [edge_index[0], loops, jnp.zeros((_PAD,), jnp.int32)])
  dst = jnp.concatenate(
      [edge_index[1], loops, jnp.full((_PAD,), _N, jnp.int32)])
  zrow1 = jnp.zeros((_SB, _FW), jnp.float32)
  zrow2 = jnp.zeros((_SB, _ROW2), jnp.float32)

  bn1 = _dense1a(x, g_bn1.reshape(1, _D), beta_bn1.reshape(1, _D))
  atti = att1[0, :, :_O1]
  attj = att1[0, :, _O1:]
  t0, t1, t2, si3, sj3 = _dense1b(bn1, W1, atti, attj)
  acc1 = _edge1(si3.reshape(_H1, _N), sj3.reshape(_H1, _N), src, dst,
                t0.reshape(_H1 * _N, _FW), t1.reshape(_H1 * _N, _FW),
                t2.reshape(_H1 * _N, _FW), zrow1)
  out1, mu3, msq3 = _dense2a(acc1, b1.reshape(1, _H1 * _O1))
  ha, hb, t_hm = _dense2b(out1, mu3.reshape(1, _H1 * _O1),
                          msq3.reshape(1, _H1 * _O1),
                          g_bn2.reshape(1, _H1 * _O1),
                          beta_bn2.reshape(1, _H1 * _O1), W2, att2)
  acc2 = _edge2(t_hm, src, dst, ha, hb, zrow2)
  return _final(acc2, b2.reshape(1, _O2))
